# baseline (speedup 1.0000x reference)
import functools

import jax
import jax.numpy as jnp
from jax import lax
from jax.experimental import pallas as pl
from jax.experimental.pallas import tpu as pltpu
from jax.experimental.pallas import tpu_sc as plsc

_L = 16
_KB = 6


@functools.lru_cache(maxsize=None)
def _make_sc_kernel(B, C, H, W):
    E = C // 2
    NC, NS = 2, 16
    NW = NC * NS
    assert H == NW and W == NW and E % _L == 0 and C == 2 * E
    KB = min(_KB, B)

    mesh = plsc.VectorSubcoreMesh(core_axis_name="c", subcore_axis_name="s")

    @functools.partial(
        pl.kernel,
        mesh=mesh,
        compiler_params=pltpu.CompilerParams(needs_layout_passes=False),
        out_type=jax.ShapeDtypeStruct((B, H, W, C), jnp.float32),
        scratch_types=[
            pltpu.VMEM((H, E), jnp.float32),
            pltpu.VMEM((W, C), jnp.float32),
            pltpu.VMEM((W, C), jnp.float32),
            pltpu.SemaphoreType.DMA,
            pltpu.SemaphoreType.DMA,
        ],
    )
    def k(row_hbm, col_hbm, out_hbm, rowt_v, slab_v, slab2_v, sem, sem2):
        s_ = lax.axis_index("s")
        c_ = lax.axis_index("c")
        row_own = c_ * NS + s_
        cp_col = pltpu.async_copy(
            col_hbm.at[pl.ds(0, W)], slab_v.at[:, pl.ds(0, E)], sem)
        cp_row = pltpu.async_copy(row_hbm.at[pl.ds(0, H)], rowt_v, sem2)
        cp_row.wait()

        def fill_right(slab, i_idx):
            iv = jnp.broadcast_to(i_idx, (_L,)).astype(jnp.int32)
            for c0 in range(0, E, _L):
                rv = plsc.load_gather(
                    rowt_v, [iv, c0 + lax.iota(jnp.int32, _L)])
                for j in range(W):
                    slab[j, pl.ds(E + c0, _L)] = rv

        fill_right(slab_v, row_own)
        cp_col.wait()
        copies = [pltpu.async_copy(slab_v, out_hbm.at[b, row_own], sem)
                  for b in range(KB)]

        @pl.when(c_ == 1)
        def _fast_core_extra():
            cp2 = pltpu.async_copy(
                col_hbm.at[pl.ds(0, W)], slab2_v.at[:, pl.ds(0, E)], sem2)
            fill_right(slab2_v, s_)
            cp2.wait()
            extra = [pltpu.async_copy(slab_v, out_hbm.at[b, row_own], sem)
                     for b in range(KB, B)]
            extra += [pltpu.async_copy(slab2_v, out_hbm.at[b, s_], sem)
                      for b in range(KB, B)]
            for cp in extra:
                cp.wait()

        for cp in copies:
            cp.wait()

    return k


def kernel(x, row_embed, col_embed):
    B, C, H, W = x.shape
    out = _make_sc_kernel(B, C, H, W)(row_embed, col_embed)
    return out.transpose(0, 3, 1, 2)


# speedup vs baseline: 1.0507x; 1.0507x over previous
import functools

import jax
import jax.numpy as jnp
from jax import lax
from jax.experimental import pallas as pl
from jax.experimental.pallas import tpu as pltpu
from jax.experimental.pallas import tpu_sc as plsc

_L = 16


@functools.lru_cache(maxsize=None)
def _make_sc_kernel(B, C, H, W):
    E = C // 2
    NC, NS = 2, 16
    NW = NC * NS
    assert H == NW and W == NW and E % _L == 0 and C == 2 * E

    mesh = plsc.VectorSubcoreMesh(core_axis_name="c", subcore_axis_name="s")

    @functools.partial(
        pl.kernel,
        mesh=mesh,
        compiler_params=pltpu.CompilerParams(needs_layout_passes=False),
        out_type=jax.ShapeDtypeStruct((B, H, W, C), jnp.float32),
        scratch_types=[
            pltpu.VMEM((H, E), jnp.float32),
            pltpu.VMEM((W, C), jnp.float32),
            pltpu.SemaphoreType.DMA,
            pltpu.SemaphoreType.DMA,
        ],
    )
    def k(row_hbm, col_hbm, out_hbm, rowt_v, slab_v, sem, sem2):
        wid = lax.axis_index("c") * NS + lax.axis_index("s")
        cp_col = pltpu.async_copy(
            col_hbm.at[pl.ds(0, W)], slab_v.at[:, pl.ds(0, E)], sem)
        cp_row = pltpu.async_copy(row_hbm.at[pl.ds(0, H)], rowt_v, sem2)
        cp_row.wait()
        wv = jnp.broadcast_to(wid, (_L,)).astype(jnp.int32)
        for c0 in range(0, E, _L):
            rv = plsc.load_gather(rowt_v, [wv, c0 + lax.iota(jnp.int32, _L)])
            for j in range(W):
                slab_v[j, pl.ds(E + c0, _L)] = rv
        cp_col.wait()
        copies = [pltpu.async_copy(slab_v, out_hbm.at[b, wid], sem)
                  for b in range(B)]
        for cp in copies:
            cp.wait()

    return k


def kernel(x, row_embed, col_embed):
    B, C, H, W = x.shape
    out = _make_sc_kernel(B, C, H, W)(row_embed, col_embed)
    return out.transpose(0, 3, 1, 2)
